# baseline (speedup 1.0000x reference)
import functools

import jax
import jax.numpy as jnp
from jax import lax
from jax.experimental import pallas as pl
from jax.experimental.pallas import tpu as pltpu
from jax.experimental.pallas import tpu_sc as plsc

B, F, N1, N2, K = 4, 256, 2048, 8192, 8
TQ = 1024
EPS = 1e-7

NC, NS, L = 2, 16, 16
NW = NC * NS
QTOT = N2
QW = QTOT // NW
CQ = 16
NCHUNK = QW // CQ


def _knn_body(xyzt_ref, q_ref, idx_ref, w_ref):
    q = q_ref[...]
    d2 = None
    for d in range(3):
        qd = q[:, d:d + 1]
        sd = xyzt_ref[d:d + 1, :]
        t = qd - sd
        d2 = t * t if d2 is None else d2 + t * t
    H = N1 // 2
    l, r = d2[:, :H], d2[:, H:]
    cmask = l <= r
    il = lax.broadcasted_iota(jnp.int32, (TQ, H), 1).astype(jnp.float32)
    P = jnp.where(cmask, l, r)
    O = jnp.where(cmask, r, l)
    idxc = jnp.where(cmask, il, il + H)
    idxo = jnp.where(cmask, il + H, il)
    big = jnp.float32(jnp.inf)
    ms, ams = [], []
    for _ in range(K):
        m = jnp.min(P, axis=1, keepdims=True)
        am = jnp.min(jnp.where(P == m, idxc, jnp.float32(N1)), axis=1,
                     keepdims=True)
        ms.append(m)
        ams.append(am)
        hit = idxc == am
        P = jnp.where(hit, O, P)
        O = jnp.where(hit, big, O)
        idxc = jnp.where(hit, idxo, idxc)
    m8 = jnp.concatenate(ms, axis=1)
    am8 = jnp.minimum(jnp.concatenate(ams, axis=1),
                      jnp.float32(N1 - 1)).astype(jnp.int32)
    dist = jnp.sqrt(jnp.maximum(m8, 0.0))
    wt = (1.0 + EPS) / (dist + EPS)
    wt = wt / jnp.sum(wt, axis=1, keepdims=True)
    idx_ref[...] = am8
    w_ref[...] = jnp.concatenate([wt, jnp.zeros((TQ, 16 - K), jnp.float32)],
                                 axis=1)


def _knn(xyz_t, xyz_up):
    return pl.pallas_call(
        _knn_body,
        grid=(N2 // TQ,),
        in_specs=[
            pl.BlockSpec((3, N1), lambda j: (0, 0)),
            pl.BlockSpec((TQ, 3), lambda j: (j, 0)),
        ],
        out_specs=[
            pl.BlockSpec((TQ, K), lambda j: (j, 0)),
            pl.BlockSpec((TQ, 16), lambda j: (j, 0)),
        ],
        out_shape=[
            jax.ShapeDtypeStruct((N2, K), jnp.int32),
            jax.ShapeDtypeStruct((N2, 16), jnp.float32),
        ],
    )(xyz_t, xyz_up)


@functools.partial(
    pl.kernel,
    mesh=plsc.VectorSubcoreMesh(core_axis_name="c", subcore_axis_name="s"),
    out_type=jax.ShapeDtypeStruct((QTOT, F), jnp.float32),
    scratch_types=[
        pltpu.VMEM((4, CQ * K), jnp.int32),
        pltpu.VMEM((4, CQ * 16), jnp.float32),
        pltpu.VMEM((2, CQ * K, F), jnp.float32),
        pltpu.VMEM((2, CQ, F), jnp.float32),
    ] + [pltpu.SemaphoreType.DMA] * 8,
)
def _combine(feat_hbm, idx_hbm, w_hbm, out_hbm, idx_v, w_v, rows_v, out_v,
             iw0, iw1, iw2, iw3, rs0, rs1, os0, os1):
    wid = lax.axis_index("s") * NC + lax.axis_index("c")
    qbase = wid * QW
    iw_sems = (iw0, iw1, iw2, iw3)
    row_sems = (rs0, rs1)
    out_sems = (os0, os1)

    def fire_iw(c, s4):
        base = qbase + c * CQ
        pltpu.async_copy(idx_hbm.at[pl.ds(base * K, CQ * K)], idx_v.at[s4],
                         iw_sems[s4])
        pltpu.async_copy(w_hbm.at[pl.ds(base * 16, CQ * 16)], w_v.at[s4],
                         iw_sems[s4])

    def wait_iw(c, s4):
        base = qbase + c * CQ
        pltpu.make_async_copy(idx_hbm.at[pl.ds(base * K, CQ * K)],
                              idx_v.at[s4], iw_sems[s4]).wait()
        pltpu.make_async_copy(w_hbm.at[pl.ds(base * 16, CQ * 16)],
                              w_v.at[s4], iw_sems[s4]).wait()

    def fire_gather(s4, s2):
        pltpu.async_copy(feat_hbm.at[idx_v.at[s4]], rows_v.at[s2],
                         row_sems[s2])

    for c in range(3):
        fire_iw(c, c)
    for c in range(2):
        wait_iw(c, c)
        fire_gather(c, c)

    def outer(c0, carry):
        for u in range(4):
            c = c0 * 4 + u
            s2, s4g, s4f = u % 2, (u + 2) % 4, (u + 3) % 4
            pltpu.make_async_copy(feat_hbm.at[idx_v.at[u]], rows_v.at[s2],
                                  row_sems[s2]).wait()

            @pl.when(c >= 2)
            def _():
                pltpu.make_async_copy(
                    out_v.at[s2],
                    out_hbm.at[pl.ds(qbase + (c - 2) * CQ, CQ)],
                    out_sems[s2]).wait()

            def qbody(q, carry2, _u=u, _s2=s2):
                qi = q * K
                wvec = w_v[_u, pl.ds(q * 16, 16)]
                wks = [wvec[k] for k in range(K)]
                for j in range(F // L):
                    sl = pl.ds(j * L, L)
                    acc = wks[0] * rows_v[_s2, qi, sl]
                    for k in range(1, K):
                        acc = acc + wks[k] * rows_v[_s2, qi + k, sl]
                    out_v[_s2, q, sl] = acc
                return carry2

            lax.fori_loop(0, CQ, qbody, 0)
            pltpu.async_copy(out_v.at[s2],
                             out_hbm.at[pl.ds(qbase + c * CQ, CQ)],
                             out_sems[s2])

            @pl.when(c + 3 < NCHUNK)
            def _():
                fire_iw(c + 3, s4f)

            @pl.when(c + 2 < NCHUNK)
            def _():
                wait_iw(c + 2, s4g)
                fire_gather(s4g, s2)
        return carry

    lax.fori_loop(0, NCHUNK // 4, outer, 0)

    for c in range(NCHUNK - 2, NCHUNK):
        pltpu.make_async_copy(out_v.at[c % 2],
                              out_hbm.at[pl.ds(qbase + c * CQ, CQ)],
                              out_sems[c % 2]).wait()


def kernel(features, xyz, xyz_upsampled):
    xyz_t = jnp.transpose(xyz, (0, 2, 1))
    feat2d = jnp.transpose(features[..., 0], (0, 2, 1))

    outs = []
    for b in range(B):
        idx, w = _knn(xyz_t[b], xyz_upsampled[b])
        outs.append(_combine(feat2d[b], idx.reshape(QTOT * K),
                             w.reshape(QTOT * 16)))
    out = jnp.stack(outs)
    return jnp.transpose(out, (0, 2, 1))[..., None]


# speedup vs baseline: 1.0226x; 1.0226x over previous
import functools

import jax
import jax.numpy as jnp
from jax import lax
from jax.experimental import pallas as pl
from jax.experimental.pallas import tpu as pltpu
from jax.experimental.pallas import tpu_sc as plsc

B, F, N1, N2, K = 4, 256, 2048, 8192, 8
TQ = 512
EPS = 1e-7

NC, NS, L = 2, 16, 16
NW = NC * NS
QTOT = N2
QW = QTOT // NW
CQ = 16
NCHUNK = QW // CQ


def _knn_body(xyzt_ref, q_ref, idx_ref, w_ref):
    q = q_ref[...]
    d2 = None
    for d in range(3):
        qd = q[:, d:d + 1]
        sd = xyzt_ref[d:d + 1, :]
        t = qd - sd
        d2 = t * t if d2 is None else d2 + t * t
    H = N1 // 2
    l, r = d2[:, :H], d2[:, H:]
    cmask = l <= r
    il = lax.broadcasted_iota(jnp.int32, (TQ, H), 1).astype(jnp.float32)
    P = jnp.where(cmask, l, r)
    O = jnp.where(cmask, r, l)
    idxc = jnp.where(cmask, il, il + H)
    idxo = jnp.where(cmask, il + H, il)
    big = jnp.float32(jnp.inf)
    ms, ams = [], []
    for _ in range(K):
        m = jnp.min(P, axis=1, keepdims=True)
        am = jnp.min(jnp.where(P == m, idxc, jnp.float32(N1)), axis=1,
                     keepdims=True)
        ms.append(m)
        ams.append(am)
        hit = idxc == am
        P = jnp.where(hit, O, P)
        O = jnp.where(hit, big, O)
        idxc = jnp.where(hit, idxo, idxc)
    m8 = jnp.concatenate(ms, axis=1)
    am8 = jnp.minimum(jnp.concatenate(ams, axis=1),
                      jnp.float32(N1 - 1)).astype(jnp.int32)
    dist = jnp.sqrt(jnp.maximum(m8, 0.0))
    wt = (1.0 + EPS) / (dist + EPS)
    wt = wt / jnp.sum(wt, axis=1, keepdims=True)
    idx_ref[...] = am8
    w_ref[...] = jnp.concatenate([wt, jnp.zeros((TQ, 16 - K), jnp.float32)],
                                 axis=1)


def _knn(xyz_t, xyz_up):
    return pl.pallas_call(
        _knn_body,
        grid=(N2 // TQ,),
        in_specs=[
            pl.BlockSpec((3, N1), lambda j: (0, 0)),
            pl.BlockSpec((TQ, 3), lambda j: (j, 0)),
        ],
        out_specs=[
            pl.BlockSpec((TQ, K), lambda j: (j, 0)),
            pl.BlockSpec((TQ, 16), lambda j: (j, 0)),
        ],
        out_shape=[
            jax.ShapeDtypeStruct((N2, K), jnp.int32),
            jax.ShapeDtypeStruct((N2, 16), jnp.float32),
        ],
    )(xyz_t, xyz_up)


@functools.partial(
    pl.kernel,
    mesh=plsc.VectorSubcoreMesh(core_axis_name="c", subcore_axis_name="s"),
    out_type=jax.ShapeDtypeStruct((QTOT, F), jnp.float32),
    scratch_types=[
        pltpu.VMEM((4, CQ * K), jnp.int32),
        pltpu.VMEM((4, CQ * 16), jnp.float32),
        pltpu.VMEM((2, CQ * K, F), jnp.float32),
        pltpu.VMEM((2, CQ, F), jnp.float32),
    ] + [pltpu.SemaphoreType.DMA] * 8,
)
def _combine(feat_hbm, idx_hbm, w_hbm, out_hbm, idx_v, w_v, rows_v, out_v,
             iw0, iw1, iw2, iw3, rs0, rs1, os0, os1):
    wid = lax.axis_index("s") * NC + lax.axis_index("c")
    qbase = wid * QW
    iw_sems = (iw0, iw1, iw2, iw3)
    row_sems = (rs0, rs1)
    out_sems = (os0, os1)

    def fire_iw(c, s4):
        base = qbase + c * CQ
        pltpu.async_copy(idx_hbm.at[pl.ds(base * K, CQ * K)], idx_v.at[s4],
                         iw_sems[s4])
        pltpu.async_copy(w_hbm.at[pl.ds(base * 16, CQ * 16)], w_v.at[s4],
                         iw_sems[s4])

    def wait_iw(c, s4):
        base = qbase + c * CQ
        pltpu.make_async_copy(idx_hbm.at[pl.ds(base * K, CQ * K)],
                              idx_v.at[s4], iw_sems[s4]).wait()
        pltpu.make_async_copy(w_hbm.at[pl.ds(base * 16, CQ * 16)],
                              w_v.at[s4], iw_sems[s4]).wait()

    def fire_gather(s4, s2):
        pltpu.async_copy(feat_hbm.at[idx_v.at[s4]], rows_v.at[s2],
                         row_sems[s2])

    for c in range(3):
        fire_iw(c, c)
    for c in range(2):
        wait_iw(c, c)
        fire_gather(c, c)

    def outer(c0, carry):
        for u in range(4):
            c = c0 * 4 + u
            s2, s4g, s4f = u % 2, (u + 2) % 4, (u + 3) % 4
            pltpu.make_async_copy(feat_hbm.at[idx_v.at[u]], rows_v.at[s2],
                                  row_sems[s2]).wait()

            @pl.when(c >= 2)
            def _():
                pltpu.make_async_copy(
                    out_v.at[s2],
                    out_hbm.at[pl.ds(qbase + (c - 2) * CQ, CQ)],
                    out_sems[s2]).wait()

            def qbody(q, carry2, _u=u, _s2=s2):
                qi = q * K
                wvec = w_v[_u, pl.ds(q * 16, 16)]
                wks = [wvec[k] for k in range(K)]
                for j in range(F // L):
                    sl = pl.ds(j * L, L)
                    acc = wks[0] * rows_v[_s2, qi, sl]
                    for k in range(1, K):
                        acc = acc + wks[k] * rows_v[_s2, qi + k, sl]
                    out_v[_s2, q, sl] = acc
                return carry2

            lax.fori_loop(0, CQ, qbody, 0)
            pltpu.async_copy(out_v.at[s2],
                             out_hbm.at[pl.ds(qbase + c * CQ, CQ)],
                             out_sems[s2])

            @pl.when(c + 3 < NCHUNK)
            def _():
                fire_iw(c + 3, s4f)

            @pl.when(c + 2 < NCHUNK)
            def _():
                wait_iw(c + 2, s4g)
                fire_gather(s4g, s2)
        return carry

    lax.fori_loop(0, NCHUNK // 4, outer, 0)

    for c in range(NCHUNK - 2, NCHUNK):
        pltpu.make_async_copy(out_v.at[c % 2],
                              out_hbm.at[pl.ds(qbase + c * CQ, CQ)],
                              out_sems[c % 2]).wait()


def kernel(features, xyz, xyz_upsampled):
    xyz_t = jnp.transpose(xyz, (0, 2, 1))
    feat2d = jnp.transpose(features[..., 0], (0, 2, 1))

    outs = []
    for b in range(B):
        idx, w = _knn(xyz_t[b], xyz_upsampled[b])
        outs.append(_combine(feat2d[b], idx.reshape(QTOT * K),
                             w.reshape(QTOT * 16)))
    out = jnp.stack(outs)
    return jnp.transpose(out, (0, 2, 1))[..., None]
